# Initial kernel scaffold; baseline (speedup 1.0000x reference)
import jax
import jax.numpy as jnp
from jax.experimental import pallas as pl


def kernel(x, w_gate, w_noise, e0_W1, e0_b1, e0_W2, e0_b2, e0_W3, e0_b3, e1_W1, e1_b1, e1_W2, e1_b2, e1_W3, e1_b3, e2_W1, e2_b1, e2_W2, e2_b2, e2_W3, e2_b3, e3_W1, e3_b1, e3_W2, e3_b2, e3_W3, e3_b3):
    raise NotImplementedError("write your pallas kernel here")


# speedup vs baseline: 1.3343x; 1.3343x over previous
import jax
import jax.numpy as jnp
from jax.experimental import pallas as pl
from jax.experimental.pallas import tpu as pltpu

B = 256
T = 64
D = 512
E = 4
BN = [32, 64, 128, 256]
GB = 32


def _gate_kernel(x_ref, wg_ref, eg_ref, loss_ref, imp_ref, load_ref):
    i = pl.program_id(0)
    nb = pl.num_programs(0)

    @pl.when(i == 0)
    def _():
        for e in range(E):
            imp_ref[e] = 0.0
            load_ref[e] = 0.0

    logits = jnp.dot(x_ref[...], wg_ref[...], preferred_element_type=jnp.float32)
    iota = jax.lax.broadcasted_iota(jnp.int32, (GB, E), 1)
    m1 = jnp.max(logits, axis=1, keepdims=True)
    e0 = jnp.min(jnp.where(logits == m1, iota, E), axis=1, keepdims=True)
    lm = jnp.where(iota == e0, -jnp.inf, logits)
    m2 = jnp.max(lm, axis=1, keepdims=True)
    e1 = jnp.min(jnp.where(lm == m2, iota, E), axis=1, keepdims=True)
    ed = jnp.exp(m2 - m1)
    g0 = 1.0 / (1.0 + ed)
    g1 = ed / (1.0 + ed)
    gates = jnp.where(iota == e0, g0, 0.0) + jnp.where(iota == e1, g1, 0.0)

    for e in range(E):
        imp_ref[e] += jnp.sum(gates[:, e])
        load_ref[e] += jnp.sum((gates[:, e] > 0.0).astype(jnp.float32))

    c8 = jax.lax.broadcasted_iota(jnp.int32, (GB, 8), 1)
    e0f = e0.astype(jnp.float32)
    e1f = e1.astype(jnp.float32)
    eg = jnp.where(c8 == 0, e0f,
         jnp.where(c8 == 1, e1f,
         jnp.where(c8 == 2, g0,
         jnp.where(c8 == 3, g1, 0.0))))
    eg_ref[...] = eg

    @pl.when(i == nb - 1)
    def _():
        eps = 1e-10
        iv = [imp_ref[e] for e in range(E)]
        lv = [load_ref[e] for e in range(E)]
        im = (iv[0] + iv[1] + iv[2] + iv[3]) / 4.0
        lmn = (lv[0] + lv[1] + lv[2] + lv[3]) / 4.0
        ivar = ((iv[0] - im) ** 2 + (iv[1] - im) ** 2 + (iv[2] - im) ** 2 + (iv[3] - im) ** 2) / 3.0
        lvar = ((lv[0] - lmn) ** 2 + (lv[1] - lmn) ** 2 + (lv[2] - lmn) ** 2 + (lv[3] - lmn) ** 2) / 3.0
        lval = (ivar / (im * im + eps) + lvar / (lmn * lmn + eps)) * 1e-2
        loss_ref[...] = jnp.full((1, 1), lval, dtype=jnp.float32)


def _gelu_exact(v):
    return 0.5 * v * (1.0 + jax.lax.erf(v * 0.7071067811865476))


def _moe_kernel(e0_ref, e1_ref, g0_ref, g1_ref, x_ref, *rest):
    wrefs = rest[:-1]
    y_ref = rest[-1]
    s = pl.program_id(0)
    e0 = e0_ref[s]
    e1 = e1_ref[s]
    g0 = g0_ref[s]
    g1 = g1_ref[s]
    xb = x_ref[0]
    ws = [jnp.where(e0 == e, g0, 0.0) + jnp.where(e1 == e, g1, 0.0) for e in range(E)]
    y_ref[0] = xb * (ws[0] + ws[1] + ws[2] + ws[3])
    for e in range(E):
        W1, b1, W2, b2, W3, b3 = wrefs[6 * e:6 * e + 6]

        @pl.when(ws[e] > 0.0)
        def _(W1=W1, b1=b1, W2=W2, b2=b2, W3=W3, b3=b3, w=ws[e]):
            h = jnp.dot(xb, W1[...], preferred_element_type=jnp.float32) + b1[...]
            h = _gelu_exact(h)
            h = jnp.dot(h, W2[...], preferred_element_type=jnp.float32) + b2[...]
            h = _gelu_exact(h)
            h = jnp.dot(h, W3[...], preferred_element_type=jnp.float32) + b3[...]
            y_ref[0] += w * h


def kernel(x, w_gate, w_noise, e0_W1, e0_b1, e0_W2, e0_b2, e0_W3, e0_b3,
           e1_W1, e1_b1, e1_W2, e1_b2, e1_W3, e1_b3,
           e2_W1, e2_b1, e2_W2, e2_b2, e2_W3, e2_b3,
           e3_W1, e3_b1, e3_W2, e3_b2, e3_W3, e3_b3):
    del w_noise

    eg, loss2 = pl.pallas_call(
        _gate_kernel,
        grid=(B // GB,),
        in_specs=[
            pl.BlockSpec((GB, T * D), lambda i: (i, 0)),
            pl.BlockSpec((T * D, E), lambda i: (0, 0)),
        ],
        out_specs=[
            pl.BlockSpec((GB, 8), lambda i: (i, 0)),
            pl.BlockSpec((1, 1), lambda i: (0, 0)),
        ],
        out_shape=[
            jax.ShapeDtypeStruct((B, 8), jnp.float32),
            jax.ShapeDtypeStruct((1, 1), jnp.float32),
        ],
        scratch_shapes=[pltpu.SMEM((E,), jnp.float32), pltpu.SMEM((E,), jnp.float32)],
        compiler_params=pltpu.CompilerParams(dimension_semantics=("arbitrary",)),
    )(x, w_gate)

    e0i = eg[:, 0].astype(jnp.int32)
    e1i = eg[:, 1].astype(jnp.int32)
    g0 = eg[:, 2]
    g1 = eg[:, 3]

    x3 = x.reshape(B, T, D)
    experts = [
        (e0_W1, e0_b1.reshape(1, -1), e0_W2, e0_b2.reshape(1, -1), e0_W3, e0_b3.reshape(1, -1)),
        (e1_W1, e1_b1.reshape(1, -1), e1_W2, e1_b2.reshape(1, -1), e1_W3, e1_b3.reshape(1, -1)),
        (e2_W1, e2_b1.reshape(1, -1), e2_W2, e2_b2.reshape(1, -1), e2_W3, e2_b3.reshape(1, -1)),
        (e3_W1, e3_b1.reshape(1, -1), e3_W2, e3_b2.reshape(1, -1), e3_W3, e3_b3.reshape(1, -1)),
    ]
    wargs = [w for ex in experts for w in ex]

    const = lambda s, *_: tuple(0 for _ in range(2))
    wspecs = []
    for ex in experts:
        for w in ex:
            wspecs.append(pl.BlockSpec(w.shape, const))

    y3 = pl.pallas_call(
        _moe_kernel,
        grid_spec=pltpu.PrefetchScalarGridSpec(
            num_scalar_prefetch=4,
            grid=(B,),
            in_specs=[pl.BlockSpec((1, T, D), lambda s, *_: (s, 0, 0))] + wspecs,
            out_specs=pl.BlockSpec((1, T, D), lambda s, *_: (s, 0, 0)),
        ),
        out_shape=jax.ShapeDtypeStruct((B, T, D), jnp.float32),
        compiler_params=pltpu.CompilerParams(dimension_semantics=("arbitrary",)),
    )(e0i, e1i, g0, g1, x3, *wargs)

    return y3.reshape(B, T * D), loss2[0, 0]
